# initial kernel scaffold (unmeasured)
import jax
import jax.numpy as jnp
from jax import lax
from jax.experimental import pallas as pl
from jax.experimental.pallas import tpu as pltpu

N_DEV = 32
B, SQ, DM = 2, 256, 512
DH = 64
H_PER = 4
ROWS = B * SQ
CHUNK = ROWS // N_DEV


def kernel(x, Wq, K_ext, V_ext, Wo):
    my = lax.axis_index("i")
    K = lax.dynamic_slice_in_dim(K_ext, my * H_PER, H_PER, axis=2)
    V = lax.dynamic_slice_in_dim(V_ext, my * H_PER, H_PER, axis=2)

    def body(x_ref, wq_ref, k_ref, v_ref, wo_ref, out_ref,
             acc_ref, stage_ref,
             rs_send_sems, rs_recv_sems, ag_send_sems, ag_recv_sems):
        me = lax.axis_index("i")

        x2 = x_ref[...].reshape(ROWS, DM)
        q = jnp.dot(x2, wq_ref[...], preferred_element_type=jnp.float32)
        q4 = q.reshape(B, SQ, H_PER, DH)

        qb = lax.broadcasted_iota(jnp.int32, (SQ, SQ), 0) // 64
        kb = lax.broadcasted_iota(jnp.int32, (SQ, SQ), 1) // 64
        mask = (qb == kb) | ((kb % 4) == (qb % 4))

        for b in range(B):
            pb = jnp.zeros((SQ, DM), jnp.float32)
            for h in range(H_PER):
                qh = q4[b, :, h, :]
                kh = k_ref[b, :, h, :]
                vh = v_ref[b, :, h, :]
                s = lax.dot_general(
                    qh, kh, (((1,), (1,)), ((), ())),
                    preferred_element_type=jnp.float32) * 0.125
                s = jnp.where(mask, s, -1e9)
                w = jnp.exp(s - jnp.max(s, axis=-1, keepdims=True))
                w = w / jnp.sum(w, axis=-1, keepdims=True)
                ctx = jnp.dot(w, vh, preferred_element_type=jnp.float32)
                pb = pb + jnp.dot(ctx, wo_ref[pl.ds(h * DH, DH), :],
                                  preferred_element_type=jnp.float32)
            acc_ref[pl.ds(b * 16, 16)] = pb.reshape(16, CHUNK, DM)

        rs_sends = []
        for off in range(1, N_DEV):
            t = lax.rem(me + off, N_DEV)
            rdma = pltpu.make_async_remote_copy(
                src_ref=acc_ref.at[t],
                dst_ref=stage_ref.at[me],
                send_sem=rs_send_sems.at[t],
                recv_sem=rs_recv_sems.at[me],
                device_id=(t,),
                device_id_type=pl.DeviceIdType.MESH,
            )
            rdma.start()
            rs_sends.append(rdma)

        red = acc_ref[me]
        for off in range(1, N_DEV):
            j = lax.rem(me + off, N_DEV)
            recv = pltpu.make_async_remote_copy(
                src_ref=acc_ref.at[j],
                dst_ref=stage_ref.at[j],
                send_sem=rs_send_sems.at[j],
                recv_sem=rs_recv_sems.at[j],
                device_id=(j,),
                device_id_type=pl.DeviceIdType.MESH,
            )
            recv.wait_recv()
            red = red + stage_ref[j]
        acc_ref[me] = red

        ag_sends = []
        for off in range(1, N_DEV):
            t = lax.rem(me + off, N_DEV)
            rdma = pltpu.make_async_remote_copy(
                src_ref=acc_ref.at[me],
                dst_ref=acc_ref.at[me],
                send_sem=ag_send_sems.at[t],
                recv_sem=ag_recv_sems.at[me],
                device_id=(t,),
                device_id_type=pl.DeviceIdType.MESH,
            )
            rdma.start()
            ag_sends.append(rdma)

        for off in range(1, N_DEV):
            j = lax.rem(me + off, N_DEV)
            recv = pltpu.make_async_remote_copy(
                src_ref=acc_ref.at[j],
                dst_ref=acc_ref.at[j],
                send_sem=ag_send_sems.at[j],
                recv_sem=ag_recv_sems.at[j],
                device_id=(j,),
                device_id_type=pl.DeviceIdType.MESH,
            )
            recv.wait_recv()

        for rdma in rs_sends + ag_sends:
            rdma.wait_send()

        out_ref[...] = acc_ref[...].reshape(B, SQ, DM)

    return pl.pallas_call(
        body,
        out_shape=jax.ShapeDtypeStruct((B, SQ, DM), jnp.float32),
        in_specs=[pl.BlockSpec(memory_space=pltpu.VMEM)] * 5,
        out_specs=pl.BlockSpec(memory_space=pltpu.VMEM),
        scratch_shapes=[
            pltpu.VMEM((N_DEV, CHUNK, DM), jnp.float32),
            pltpu.VMEM((N_DEV, CHUNK, DM), jnp.float32),
            pltpu.SemaphoreType.DMA((N_DEV,)),
            pltpu.SemaphoreType.DMA((N_DEV,)),
            pltpu.SemaphoreType.DMA((N_DEV,)),
            pltpu.SemaphoreType.DMA((N_DEV,)),
        ],
        compiler_params=pltpu.CompilerParams(collective_id=0),
    )(x, Wq, K, V, Wo)


# baseline (device time: 84300 ns/iter reference)
import jax
import jax.numpy as jnp
from jax import lax
from jax.experimental import pallas as pl
from jax.experimental.pallas import tpu as pltpu

N_DEV = 32
B, SQ, DM = 2, 256, 512
DH = 64
H_PER = 4
ROWS = B * SQ
CHUNK = ROWS // N_DEV


def kernel(x, Wq, K_ext, V_ext, Wo):
    my = lax.axis_index("i")
    K = lax.dynamic_slice_in_dim(K_ext, my * H_PER, H_PER, axis=2)
    V = lax.dynamic_slice_in_dim(V_ext, my * H_PER, H_PER, axis=2)

    def body(x_ref, wq_ref, k_ref, v_ref, wo_ref, out_ref,
             acc_ref, stage_ref,
             rs_send_sems, rs_recv_sems, ag_send_sems, ag_recv_sems):
        me = lax.axis_index("i")

        x2 = x_ref[...].reshape(ROWS, DM)
        q = jnp.dot(x2, wq_ref[...], preferred_element_type=jnp.float32)
        q4 = q.reshape(B, SQ, H_PER, DH)

        qb = lax.broadcasted_iota(jnp.int32, (SQ, SQ), 0) // 64
        kb = lax.broadcasted_iota(jnp.int32, (SQ, SQ), 1) // 64
        mask = (qb == kb) | ((kb % 4) == (qb % 4))

        for b in range(B):
            pb = jnp.zeros((SQ, DM), jnp.float32)
            for h in range(H_PER):
                qh = q4[b, :, h, :]
                kh = k_ref[b, :, h, :]
                vh = v_ref[b, :, h, :]
                s = lax.dot_general(
                    qh, kh, (((1,), (1,)), ((), ())),
                    preferred_element_type=jnp.float32) * 0.125
                s = jnp.where(mask, s, -1e9)
                w = jnp.exp(s - jnp.max(s, axis=-1, keepdims=True))
                w = w / jnp.sum(w, axis=-1, keepdims=True)
                ctx = jnp.dot(w, vh, preferred_element_type=jnp.float32)
                pb = pb + jnp.dot(ctx, wo_ref[pl.ds(h * DH, DH), :],
                                  preferred_element_type=jnp.float32)
            acc_ref[pl.ds(b * 16, 16)] = pb.reshape(16, CHUNK, DM)

        rs_sends = []
        for off in range(1, N_DEV):
            t = lax.rem(me + off, N_DEV)
            rdma = pltpu.make_async_remote_copy(
                src_ref=acc_ref.at[t],
                dst_ref=stage_ref.at[me],
                send_sem=rs_send_sems.at[t],
                recv_sem=rs_recv_sems.at[me],
                device_id=(t,),
                device_id_type=pl.DeviceIdType.MESH,
            )
            rdma.start()
            rs_sends.append(rdma)

        red = acc_ref[pl.ds(me, 1)]
        for off in range(1, N_DEV):
            j = lax.rem(me + off, N_DEV)
            recv = pltpu.make_async_remote_copy(
                src_ref=acc_ref.at[j],
                dst_ref=stage_ref.at[j],
                send_sem=rs_send_sems.at[j],
                recv_sem=rs_recv_sems.at[j],
                device_id=(j,),
                device_id_type=pl.DeviceIdType.MESH,
            )
            recv.wait_recv()
            red = red + stage_ref[pl.ds(j, 1)]
        acc_ref[pl.ds(me, 1)] = red

        ag_sends = []
        for off in range(1, N_DEV):
            t = lax.rem(me + off, N_DEV)
            rdma = pltpu.make_async_remote_copy(
                src_ref=acc_ref.at[me],
                dst_ref=acc_ref.at[me],
                send_sem=ag_send_sems.at[t],
                recv_sem=ag_recv_sems.at[me],
                device_id=(t,),
                device_id_type=pl.DeviceIdType.MESH,
            )
            rdma.start()
            ag_sends.append(rdma)

        for off in range(1, N_DEV):
            j = lax.rem(me + off, N_DEV)
            recv = pltpu.make_async_remote_copy(
                src_ref=acc_ref.at[j],
                dst_ref=acc_ref.at[j],
                send_sem=ag_send_sems.at[j],
                recv_sem=ag_recv_sems.at[j],
                device_id=(j,),
                device_id_type=pl.DeviceIdType.MESH,
            )
            recv.wait_recv()

        for rdma in rs_sends + ag_sends:
            rdma.wait_send()

        out_ref[...] = acc_ref[...].reshape(B, SQ, DM)

    return pl.pallas_call(
        body,
        out_shape=jax.ShapeDtypeStruct((B, SQ, DM), jnp.float32),
        in_specs=[pl.BlockSpec(memory_space=pltpu.VMEM)] * 5,
        out_specs=pl.BlockSpec(memory_space=pltpu.VMEM),
        scratch_shapes=[
            pltpu.VMEM((N_DEV, CHUNK, DM), jnp.float32),
            pltpu.VMEM((N_DEV, CHUNK, DM), jnp.float32),
            pltpu.SemaphoreType.DMA((N_DEV,)),
            pltpu.SemaphoreType.DMA((N_DEV,)),
            pltpu.SemaphoreType.DMA((N_DEV,)),
            pltpu.SemaphoreType.DMA((N_DEV,)),
        ],
    )(x, Wq, K, V, Wo)
